# baseline (speedup 1.0000x reference)
import jax
import jax.numpy as jnp
from jax import lax
from jax.experimental import pallas as pl
from jax.experimental.pallas import tpu as pltpu
from jax.experimental.pallas import tpu_sc as plsc

N = 10000
E = 320000
D = 128

NC = 2
NS = 16
HD = D // NC
CH = 128
ET = E // NS
NCHUNK = 160
PADW = NCHUNK * CH - ET
NPAD = N + 8
DEGW = 8

ROWS_BIG = 640
ROWS_LAST = N - 15 * ROWS_BIG

_sc_mesh = plsc.VectorSubcoreMesh(core_axis_name="c", subcore_axis_name="s")


def _segsum_body(with_deg, a_hbm, src_hbm, dst_hbm, zr_hbm, zd_hbm, ones_hbm,
                 rows_out, deg_out, src_v, dst_v, rb0, rb1, rb2, rb3,
                 sg0, sg1, sg2, sg3, ss0, ss1, ss2, ss3,
                 ones_v, acc_sh, deg_sh):
    c = lax.axis_index("c")
    s = lax.axis_index("s")

    @pl.when(s < NS - 1)
    def _():
        pltpu.sync_copy(zr_hbm, acc_sh.at[pl.ds(s * ROWS_BIG, ROWS_BIG)])
        if with_deg:
            pltpu.sync_copy(zd_hbm, deg_sh.at[pl.ds(s * ROWS_BIG, ROWS_BIG)])

    @pl.when(s == NS - 1)
    def _():
        pltpu.sync_copy(zr_hbm.at[pl.ds(0, ROWS_LAST)],
                        acc_sh.at[pl.ds(15 * ROWS_BIG, ROWS_LAST)])
        if with_deg:
            pltpu.sync_copy(zd_hbm.at[pl.ds(0, ROWS_LAST)],
                            deg_sh.at[pl.ds(15 * ROWS_BIG, ROWS_LAST)])

    pltpu.sync_copy(src_hbm.at[s], src_v)
    pltpu.sync_copy(dst_hbm.at[s], dst_v)
    if with_deg:
        pltpu.sync_copy(ones_hbm, ones_v)

    plsc.subcore_barrier()

    table = a_hbm.at[c]
    RB = (rb0, rb1, rb2, rb3)
    SG = (sg0, sg1, sg2, sg3)
    SS = (ss0, ss1, ss2, ss3)

    def gather_start(g, j):
        pltpu.async_copy(table.at[src_v.at[g]], RB[j], SG[j])

    def gather_wait(g, j):
        pltpu.make_async_copy(table.at[src_v.at[g]], RB[j], SG[j]).wait()

    def scat_start(g, j):
        pltpu.async_copy(RB[j], acc_sh.at[dst_v.at[g]], SS[j], add=True)
        if with_deg:
            @pl.when(c == 0)
            def _():
                pltpu.sync_copy(ones_v, deg_sh.at[dst_v.at[g]], add=True)

    def scat_wait(g, j):
        pltpu.make_async_copy(RB[j], acc_sh.at[dst_v.at[g]], SS[j]).wait()

    def ring(i, carry):
        for j in range(4):
            g = i * 4 + j

            @pl.when(i > 0)
            def _():
                scat_wait(g - 4, j)
            gather_start(g, j)
            jm = (j + 2) % 4
            if j < 2:
                @pl.when(i > 0)
                def _():
                    gather_wait(g - 2, jm)
                    scat_start(g - 2, jm)
            else:
                gather_wait(g - 2, jm)
                scat_start(g - 2, jm)
        return carry

    lax.fori_loop(0, NCHUNK // 4, ring, 0)

    for g in (NCHUNK - 2, NCHUNK - 1):
        gather_wait(g, g % 4)
        scat_start(g, g % 4)
    for g in range(NCHUNK - 4, NCHUNK):
        scat_wait(g, g % 4)

    plsc.subcore_barrier()

    @pl.when(s < NS - 1)
    def _():
        r0 = s * ROWS_BIG
        pltpu.sync_copy(acc_sh.at[pl.ds(r0, ROWS_BIG)],
                        rows_out.at[c].at[pl.ds(r0, ROWS_BIG)])
        if with_deg:
            @pl.when(c == 0)
            def _():
                pltpu.sync_copy(deg_sh.at[pl.ds(r0, ROWS_BIG)],
                                deg_out.at[pl.ds(r0, ROWS_BIG)])

    @pl.when(s == NS - 1)
    def _():
        r0 = 15 * ROWS_BIG
        pltpu.sync_copy(acc_sh.at[pl.ds(r0, ROWS_LAST)],
                        rows_out.at[c].at[pl.ds(r0, ROWS_LAST)])
        if with_deg:
            @pl.when(c == 0)
            def _():
                pltpu.sync_copy(deg_sh.at[pl.ds(r0, ROWS_LAST)],
                                deg_out.at[pl.ds(r0, ROWS_LAST)])


def _make_segsum(with_deg):
    out_type = [jax.ShapeDtypeStruct((NC, N, HD), jnp.float32)]
    if with_deg:
        out_type.append(jax.ShapeDtypeStruct((N, DEGW), jnp.float32))
    scratch = [
        pltpu.VMEM((NCHUNK, CH), jnp.int32),
        pltpu.VMEM((NCHUNK, CH), jnp.int32),
        pltpu.VMEM((CH, HD), jnp.float32),
        pltpu.VMEM((CH, HD), jnp.float32),
        pltpu.VMEM((CH, HD), jnp.float32),
        pltpu.VMEM((CH, HD), jnp.float32),
        pltpu.SemaphoreType.DMA,
        pltpu.SemaphoreType.DMA,
        pltpu.SemaphoreType.DMA,
        pltpu.SemaphoreType.DMA,
        pltpu.SemaphoreType.DMA,
        pltpu.SemaphoreType.DMA,
        pltpu.SemaphoreType.DMA,
        pltpu.SemaphoreType.DMA,
        pltpu.VMEM((CH, DEGW), jnp.float32),
        pltpu.VMEM_SHARED((NPAD, HD), jnp.float32),
        pltpu.VMEM_SHARED((NPAD, DEGW), jnp.float32),
    ]

    def body(a_hbm, src_hbm, dst_hbm, zr_hbm, zd_hbm, ones_hbm, *refs):
        if with_deg:
            rows_out, deg_out = refs[0], refs[1]
            rest = refs[2:]
        else:
            rows_out, deg_out = refs[0], None
            rest = refs[1:]
        _segsum_body(with_deg, a_hbm, src_hbm, dst_hbm, zr_hbm, zd_hbm,
                     ones_hbm, rows_out, deg_out, *rest)

    return pl.kernel(body,
                     out_type=tuple(out_type) if with_deg else out_type[0],
                     mesh=_sc_mesh, scratch_types=scratch,
                     compiler_params=pltpu.CompilerParams(
                         use_tc_tiling_on_sc=False))


_segsum_deg = _make_segsum(True)
_segsum = _make_segsum(False)


_RB = 1000
_GRID = N // _RB


def _fold_body(w2_ref, w3_ref, b3_ref, lw_ref, lb_ref, w32p_ref, bp_ref):
    lw = lw_ref[0]
    w32p_ref[0, :, :D] = jnp.dot(w3_ref[0], lw,
                                 preferred_element_type=jnp.float32)
    w32p_ref[0, :, D:] = jnp.dot(w2_ref[0], lw,
                                 preferred_element_type=jnp.float32)
    bp_ref[0] = (jnp.dot(b3_ref[0], lw, preferred_element_type=jnp.float32)
                 + lb_ref[0])


def _fold(W2s, W3s, b3s, lWs, lbs):
    return pl.pallas_call(
        _fold_body,
        grid=(3,),
        in_specs=[pl.BlockSpec((1, D, D), lambda i: (i, 0, 0)),
                  pl.BlockSpec((1, D, D), lambda i: (i, 0, 0)),
                  pl.BlockSpec((1, 1, D), lambda i: (i, 0, 0)),
                  pl.BlockSpec((1, D, D), lambda i: (i, 0, 0)),
                  pl.BlockSpec((1, 1, D), lambda i: (i, 0, 0))],
        out_specs=[pl.BlockSpec((1, D, 2 * D), lambda i: (i, 0, 0)),
                   pl.BlockSpec((1, 1, D), lambda i: (i, 0, 0))],
        out_shape=[jax.ShapeDtypeStruct((3, D, 2 * D), jnp.float32),
                   jax.ShapeDtypeStruct((3, 1, D), jnp.float32)],
    )(W2s, W3s, b3s, lWs, lbs)


def _midA_body(x_ref, w_ref, o_ref):
    o_ref[...] = jnp.dot(x_ref[...], w_ref[...],
                         preferred_element_type=jnp.float32)


def _midA(x, w32p):
    return pl.pallas_call(
        _midA_body,
        grid=(_GRID,),
        in_specs=[pl.BlockSpec((_RB, D), lambda i: (i, 0)),
                  pl.BlockSpec((D, 2 * D), lambda i: (0, 0))],
        out_specs=pl.BlockSpec((_RB, 2 * D), lambda i: (i, 0)),
        out_shape=jax.ShapeDtypeStruct((N, 2 * D), jnp.float32),
    )(x, w32p)


def _split_cols(t, o_ref):
    o_ref[0] = t[:, :HD]
    o_ref[1] = t[:, HD:]


def _pre_body(x_ref, w_ref, b_ref, o_ref):
    t = (jnp.dot(x_ref[...], w_ref[...], preferred_element_type=jnp.float32)
         + b_ref[...])
    _split_cols(t, o_ref)


def _pre(x, W1, b1):
    return pl.pallas_call(
        _pre_body,
        grid=(_GRID,),
        in_specs=[pl.BlockSpec((_RB, D), lambda i: (i, 0)),
                  pl.BlockSpec((D, D), lambda i: (0, 0)),
                  pl.BlockSpec((1, D), lambda i: (0, 0))],
        out_specs=pl.BlockSpec((NC, _RB, HD), lambda i: (0, i, 0)),
        out_shape=jax.ShapeDtypeStruct((NC, N, HD), jnp.float32),
    )(x, W1, b1.reshape(1, D))


def _mid_body(s_ref, t_ref, deg_ref, lw_ref, bp_ref, w1n_ref, b1n_ref,
              h_ref, a_ref):
    sblk = jnp.concatenate([s_ref[0], s_ref[1]], axis=1)
    t32 = t_ref[...]
    d = deg_ref[...]
    t = jnp.dot(sblk, lw_ref[...], preferred_element_type=jnp.float32)
    t = t + t32[:, :D] - d * t32[:, D:]
    h = jnp.maximum(t + bp_ref[...], 0.0)
    h_ref[...] = h
    a = (jnp.dot(h, w1n_ref[...], preferred_element_type=jnp.float32)
         + b1n_ref[...])
    _split_cols(a, a_ref)


def _mid(S, t32, deg1, lW, bp, W1n, b1n):
    return pl.pallas_call(
        _mid_body,
        grid=(_GRID,),
        in_specs=[pl.BlockSpec((NC, _RB, HD), lambda i: (0, i, 0)),
                  pl.BlockSpec((_RB, 2 * D), lambda i: (i, 0)),
                  pl.BlockSpec((_RB, 1), lambda i: (i, 0)),
                  pl.BlockSpec((D, D), lambda i: (0, 0)),
                  pl.BlockSpec((1, D), lambda i: (0, 0)),
                  pl.BlockSpec((D, D), lambda i: (0, 0)),
                  pl.BlockSpec((1, D), lambda i: (0, 0))],
        out_specs=[pl.BlockSpec((_RB, D), lambda i: (i, 0)),
                   pl.BlockSpec((NC, _RB, HD), lambda i: (0, i, 0))],
        out_shape=[jax.ShapeDtypeStruct((N, D), jnp.float32),
                   jax.ShapeDtypeStruct((NC, N, HD), jnp.float32)],
    )(S, t32, deg1, lW, bp, W1n, b1n.reshape(1, D))


def _post_body(s_ref, t_ref, deg_ref, lw_ref, bp_ref, h_ref):
    sblk = jnp.concatenate([s_ref[0], s_ref[1]], axis=1)
    t32 = t_ref[...]
    d = deg_ref[...]
    t = jnp.dot(sblk, lw_ref[...], preferred_element_type=jnp.float32)
    t = t + t32[:, :D] - d * t32[:, D:]
    h_ref[...] = jnp.maximum(t + bp_ref[...], 0.0)


def _post(S, t32, deg1, lW, bp):
    return pl.pallas_call(
        _post_body,
        grid=(_GRID,),
        in_specs=[pl.BlockSpec((NC, _RB, HD), lambda i: (0, i, 0)),
                  pl.BlockSpec((_RB, 2 * D), lambda i: (i, 0)),
                  pl.BlockSpec((_RB, 1), lambda i: (i, 0)),
                  pl.BlockSpec((D, D), lambda i: (0, 0)),
                  pl.BlockSpec((1, D), lambda i: (0, 0))],
        out_specs=pl.BlockSpec((_RB, D), lambda i: (i, 0)),
        out_shape=jax.ShapeDtypeStruct((N, D), jnp.float32),
    )(S, t32, deg1, lW, bp)


def kernel(x, edge_index, c1_W1, c1_b1, c1_W2, c1_W3, c1_b3, l1_W, l1_b,
           c2_W1, c2_b1, c2_W2, c2_W3, c2_b3, l2_W, l2_b,
           c3_W1, c3_b1, c3_W2, c3_W3, c3_b3, l3_W, l3_b):
    src = edge_index[0]
    dst = edge_index[1]

    srcw = src.reshape(NS, ET)
    dstw = dst.reshape(NS, ET)
    padi = jnp.arange(NS * PADW, dtype=jnp.int32).reshape(NS, PADW)
    ps = (padi * 131) % N
    pd = N + (padi % 8)
    srcp = jnp.concatenate([srcw, ps], axis=1).reshape(NS, NCHUNK, CH)
    dstp = jnp.concatenate([dstw, pd], axis=1).reshape(NS, NCHUNK, CH)

    zr = jnp.zeros((ROWS_BIG, HD), jnp.float32)
    zd = jnp.zeros((ROWS_BIG, DEGW), jnp.float32)
    ones = jnp.ones((CH, DEGW), jnp.float32)

    w32p, bps = _fold(jnp.stack([c1_W2, c2_W2, c3_W2]),
                      jnp.stack([c1_W3, c2_W3, c3_W3]),
                      jnp.stack([c1_b3, c2_b3, c3_b3]).reshape(3, 1, D),
                      jnp.stack([l1_W, l2_W, l3_W]),
                      jnp.stack([l1_b, l2_b, l3_b]).reshape(3, 1, D))

    a1 = _pre(x, c1_W1, c1_b1)
    S1, deg = _segsum_deg(a1, srcp, dstp, zr, zd, ones)
    t1 = _midA(x, w32p[0])
    deg1 = deg[:, :1]
    h1, a2 = _mid(S1, t1, deg1, l1_W, bps[0], c2_W1, c2_b1)
    S2 = _segsum(a2, srcp, dstp, zr, zd, ones)
    t2 = _midA(h1, w32p[1])
    h2, a3 = _mid(S2, t2, deg1, l2_W, bps[1], c3_W1, c3_b1)
    S3 = _segsum(a3, srcp, dstp, zr, zd, ones)
    t3 = _midA(h2, w32p[2])
    h3 = _post(S3, t3, deg1, l3_W, bps[2])
    return h3


# speedup vs baseline: 1.0377x; 1.0377x over previous
import jax
import jax.numpy as jnp
from jax import lax
from jax.experimental import pallas as pl
from jax.experimental.pallas import tpu as pltpu
from jax.experimental.pallas import tpu_sc as plsc

N = 10000
E = 320000
D = 128

NC = 2
NS = 16
HD = D // NC
CH = 125
NCHUNK = 160
NPAD = N
DEGW = 8

ROWS_BIG = 640
ROWS_LAST = N - 15 * ROWS_BIG

_sc_mesh = plsc.VectorSubcoreMesh(core_axis_name="c", subcore_axis_name="s")


def _segsum_body(with_deg, a_hbm, ei_hbm, zr_hbm, zd_hbm, ones_hbm,
                 rows_out, deg_out, src_v, dst_v, rb0, rb1, rb2, rb3,
                 sg0, sg1, sg2, sg3, ss0, ss1, ss2, ss3,
                 ones_v, acc_sh, deg_sh):
    c = lax.axis_index("c")
    s = lax.axis_index("s")

    @pl.when(s < NS - 1)
    def _():
        pltpu.sync_copy(zr_hbm, acc_sh.at[pl.ds(s * ROWS_BIG, ROWS_BIG)])
        if with_deg:
            pltpu.sync_copy(zd_hbm, deg_sh.at[pl.ds(s * ROWS_BIG, ROWS_BIG)])

    @pl.when(s == NS - 1)
    def _():
        pltpu.sync_copy(zr_hbm.at[pl.ds(0, ROWS_LAST)],
                        acc_sh.at[pl.ds(15 * ROWS_BIG, ROWS_LAST)])
        if with_deg:
            pltpu.sync_copy(zd_hbm.at[pl.ds(0, ROWS_LAST)],
                            deg_sh.at[pl.ds(15 * ROWS_BIG, ROWS_LAST)])

    pltpu.sync_copy(ei_hbm.at[0].at[pl.ds(s * NCHUNK, NCHUNK)], src_v)
    pltpu.sync_copy(ei_hbm.at[1].at[pl.ds(s * NCHUNK, NCHUNK)], dst_v)
    if with_deg:
        pltpu.sync_copy(ones_hbm, ones_v)

    plsc.subcore_barrier()

    table = a_hbm.at[c]
    RB = (rb0, rb1, rb2, rb3)
    SG = (sg0, sg1, sg2, sg3)
    SS = (ss0, ss1, ss2, ss3)

    def gather_start(g, j):
        pltpu.async_copy(table.at[src_v.at[g]], RB[j], SG[j])

    def gather_wait(g, j):
        pltpu.make_async_copy(table.at[src_v.at[g]], RB[j], SG[j]).wait()

    def scat_start(g, j):
        pltpu.async_copy(RB[j], acc_sh.at[dst_v.at[g]], SS[j], add=True)
        if with_deg:
            @pl.when(c == 0)
            def _():
                pltpu.sync_copy(ones_v, deg_sh.at[dst_v.at[g]], add=True)

    def scat_wait(g, j):
        pltpu.make_async_copy(RB[j], acc_sh.at[dst_v.at[g]], SS[j]).wait()

    def ring(i, carry):
        for j in range(4):
            g = i * 4 + j

            @pl.when(i > 0)
            def _():
                scat_wait(g - 4, j)
            gather_start(g, j)
            jm = (j + 2) % 4
            if j < 2:
                @pl.when(i > 0)
                def _():
                    gather_wait(g - 2, jm)
                    scat_start(g - 2, jm)
            else:
                gather_wait(g - 2, jm)
                scat_start(g - 2, jm)
        return carry

    lax.fori_loop(0, NCHUNK // 4, ring, 0)

    for g in (NCHUNK - 2, NCHUNK - 1):
        gather_wait(g, g % 4)
        scat_start(g, g % 4)
    for g in range(NCHUNK - 4, NCHUNK):
        scat_wait(g, g % 4)

    plsc.subcore_barrier()

    @pl.when(s < NS - 1)
    def _():
        r0 = s * ROWS_BIG
        pltpu.sync_copy(acc_sh.at[pl.ds(r0, ROWS_BIG)],
                        rows_out.at[c].at[pl.ds(r0, ROWS_BIG)])
        if with_deg:
            @pl.when(c == 0)
            def _():
                pltpu.sync_copy(deg_sh.at[pl.ds(r0, ROWS_BIG)],
                                deg_out.at[pl.ds(r0, ROWS_BIG)])

    @pl.when(s == NS - 1)
    def _():
        r0 = 15 * ROWS_BIG
        pltpu.sync_copy(acc_sh.at[pl.ds(r0, ROWS_LAST)],
                        rows_out.at[c].at[pl.ds(r0, ROWS_LAST)])
        if with_deg:
            @pl.when(c == 0)
            def _():
                pltpu.sync_copy(deg_sh.at[pl.ds(r0, ROWS_LAST)],
                                deg_out.at[pl.ds(r0, ROWS_LAST)])


def _make_segsum(with_deg):
    out_type = [jax.ShapeDtypeStruct((NC, N, HD), jnp.float32)]
    if with_deg:
        out_type.append(jax.ShapeDtypeStruct((N, DEGW), jnp.float32))
    scratch = [
        pltpu.VMEM((NCHUNK, CH), jnp.int32),
        pltpu.VMEM((NCHUNK, CH), jnp.int32),
        pltpu.VMEM((CH, HD), jnp.float32),
        pltpu.VMEM((CH, HD), jnp.float32),
        pltpu.VMEM((CH, HD), jnp.float32),
        pltpu.VMEM((CH, HD), jnp.float32),
        pltpu.SemaphoreType.DMA,
        pltpu.SemaphoreType.DMA,
        pltpu.SemaphoreType.DMA,
        pltpu.SemaphoreType.DMA,
        pltpu.SemaphoreType.DMA,
        pltpu.SemaphoreType.DMA,
        pltpu.SemaphoreType.DMA,
        pltpu.SemaphoreType.DMA,
        pltpu.VMEM((CH, DEGW), jnp.float32),
        pltpu.VMEM_SHARED((NPAD, HD), jnp.float32),
        pltpu.VMEM_SHARED((NPAD, DEGW), jnp.float32),
    ]

    def body(a_hbm, ei_hbm, zr_hbm, zd_hbm, ones_hbm, *refs):
        if with_deg:
            rows_out, deg_out = refs[0], refs[1]
            rest = refs[2:]
        else:
            rows_out, deg_out = refs[0], None
            rest = refs[1:]
        _segsum_body(with_deg, a_hbm, ei_hbm, zr_hbm, zd_hbm,
                     ones_hbm, rows_out, deg_out, *rest)

    return pl.kernel(body,
                     out_type=tuple(out_type) if with_deg else out_type[0],
                     mesh=_sc_mesh, scratch_types=scratch,
                     compiler_params=pltpu.CompilerParams(
                         use_tc_tiling_on_sc=False))


_segsum_deg = _make_segsum(True)
_segsum = _make_segsum(False)


_RB = 1000
_GRID = N // _RB


def _fold_body(w2_ref, w3_ref, b3_ref, lw_ref, lb_ref, w32p_ref, bp_ref):
    lw = lw_ref[0]
    w32p_ref[0, :, :D] = jnp.dot(w3_ref[0], lw,
                                 preferred_element_type=jnp.float32)
    w32p_ref[0, :, D:] = jnp.dot(w2_ref[0], lw,
                                 preferred_element_type=jnp.float32)
    bp_ref[0] = (jnp.dot(b3_ref[0], lw, preferred_element_type=jnp.float32)
                 + lb_ref[0])


def _fold(W2s, W3s, b3s, lWs, lbs):
    return pl.pallas_call(
        _fold_body,
        grid=(3,),
        in_specs=[pl.BlockSpec((1, D, D), lambda i: (i, 0, 0)),
                  pl.BlockSpec((1, D, D), lambda i: (i, 0, 0)),
                  pl.BlockSpec((1, 1, D), lambda i: (i, 0, 0)),
                  pl.BlockSpec((1, D, D), lambda i: (i, 0, 0)),
                  pl.BlockSpec((1, 1, D), lambda i: (i, 0, 0))],
        out_specs=[pl.BlockSpec((1, D, 2 * D), lambda i: (i, 0, 0)),
                   pl.BlockSpec((1, 1, D), lambda i: (i, 0, 0))],
        out_shape=[jax.ShapeDtypeStruct((3, D, 2 * D), jnp.float32),
                   jax.ShapeDtypeStruct((3, 1, D), jnp.float32)],
    )(W2s, W3s, b3s, lWs, lbs)


def _midA_body(x_ref, w_ref, o_ref):
    o_ref[...] = jnp.dot(x_ref[...], w_ref[...],
                         preferred_element_type=jnp.float32)


def _midA(x, w32p):
    return pl.pallas_call(
        _midA_body,
        grid=(_GRID,),
        in_specs=[pl.BlockSpec((_RB, D), lambda i: (i, 0)),
                  pl.BlockSpec((D, 2 * D), lambda i: (0, 0))],
        out_specs=pl.BlockSpec((_RB, 2 * D), lambda i: (i, 0)),
        out_shape=jax.ShapeDtypeStruct((N, 2 * D), jnp.float32),
    )(x, w32p)


def _split_cols(t, o_ref):
    o_ref[0] = t[:, :HD]
    o_ref[1] = t[:, HD:]


def _pre_body(x_ref, w_ref, b_ref, o_ref):
    t = (jnp.dot(x_ref[...], w_ref[...], preferred_element_type=jnp.float32)
         + b_ref[...])
    _split_cols(t, o_ref)


def _pre(x, W1, b1):
    return pl.pallas_call(
        _pre_body,
        grid=(_GRID,),
        in_specs=[pl.BlockSpec((_RB, D), lambda i: (i, 0)),
                  pl.BlockSpec((D, D), lambda i: (0, 0)),
                  pl.BlockSpec((1, D), lambda i: (0, 0))],
        out_specs=pl.BlockSpec((NC, _RB, HD), lambda i: (0, i, 0)),
        out_shape=jax.ShapeDtypeStruct((NC, N, HD), jnp.float32),
    )(x, W1, b1.reshape(1, D))


def _mid_body(s_ref, t_ref, deg_ref, lw_ref, bp_ref, w1n_ref, b1n_ref,
              h_ref, a_ref):
    sblk = jnp.concatenate([s_ref[0], s_ref[1]], axis=1)
    t32 = t_ref[...]
    d = deg_ref[...]
    t = jnp.dot(sblk, lw_ref[...], preferred_element_type=jnp.float32)
    t = t + t32[:, :D] - d * t32[:, D:]
    h = jnp.maximum(t + bp_ref[...], 0.0)
    h_ref[...] = h
    a = (jnp.dot(h, w1n_ref[...], preferred_element_type=jnp.float32)
         + b1n_ref[...])
    _split_cols(a, a_ref)


def _mid(S, t32, deg1, lW, bp, W1n, b1n):
    return pl.pallas_call(
        _mid_body,
        grid=(_GRID,),
        in_specs=[pl.BlockSpec((NC, _RB, HD), lambda i: (0, i, 0)),
                  pl.BlockSpec((_RB, 2 * D), lambda i: (i, 0)),
                  pl.BlockSpec((_RB, 1), lambda i: (i, 0)),
                  pl.BlockSpec((D, D), lambda i: (0, 0)),
                  pl.BlockSpec((1, D), lambda i: (0, 0)),
                  pl.BlockSpec((D, D), lambda i: (0, 0)),
                  pl.BlockSpec((1, D), lambda i: (0, 0))],
        out_specs=[pl.BlockSpec((_RB, D), lambda i: (i, 0)),
                   pl.BlockSpec((NC, _RB, HD), lambda i: (0, i, 0))],
        out_shape=[jax.ShapeDtypeStruct((N, D), jnp.float32),
                   jax.ShapeDtypeStruct((NC, N, HD), jnp.float32)],
    )(S, t32, deg1, lW, bp, W1n, b1n.reshape(1, D))


def _post_body(s_ref, t_ref, deg_ref, lw_ref, bp_ref, h_ref):
    sblk = jnp.concatenate([s_ref[0], s_ref[1]], axis=1)
    t32 = t_ref[...]
    d = deg_ref[...]
    t = jnp.dot(sblk, lw_ref[...], preferred_element_type=jnp.float32)
    t = t + t32[:, :D] - d * t32[:, D:]
    h_ref[...] = jnp.maximum(t + bp_ref[...], 0.0)


def _post(S, t32, deg1, lW, bp):
    return pl.pallas_call(
        _post_body,
        grid=(_GRID,),
        in_specs=[pl.BlockSpec((NC, _RB, HD), lambda i: (0, i, 0)),
                  pl.BlockSpec((_RB, 2 * D), lambda i: (i, 0)),
                  pl.BlockSpec((_RB, 1), lambda i: (i, 0)),
                  pl.BlockSpec((D, D), lambda i: (0, 0)),
                  pl.BlockSpec((1, D), lambda i: (0, 0))],
        out_specs=pl.BlockSpec((_RB, D), lambda i: (i, 0)),
        out_shape=jax.ShapeDtypeStruct((N, D), jnp.float32),
    )(S, t32, deg1, lW, bp)


def kernel(x, edge_index, c1_W1, c1_b1, c1_W2, c1_W3, c1_b3, l1_W, l1_b,
           c2_W1, c2_b1, c2_W2, c2_W3, c2_b3, l2_W, l2_b,
           c3_W1, c3_b1, c3_W2, c3_W3, c3_b3, l3_W, l3_b):
    eir = edge_index.reshape(2, NS * NCHUNK, CH)

    zr = jnp.zeros((ROWS_BIG, HD), jnp.float32)
    zd = jnp.zeros((ROWS_BIG, DEGW), jnp.float32)
    ones = jnp.ones((CH, DEGW), jnp.float32)

    w32p, bps = _fold(jnp.stack([c1_W2, c2_W2, c3_W2]),
                      jnp.stack([c1_W3, c2_W3, c3_W3]),
                      jnp.stack([c1_b3, c2_b3, c3_b3]).reshape(3, 1, D),
                      jnp.stack([l1_W, l2_W, l3_W]),
                      jnp.stack([l1_b, l2_b, l3_b]).reshape(3, 1, D))

    a1 = _pre(x, c1_W1, c1_b1)
    S1, deg = _segsum_deg(a1, eir, zr, zd, ones)
    t1 = _midA(x, w32p[0])
    deg1 = deg[:, :1]
    h1, a2 = _mid(S1, t1, deg1, l1_W, bps[0], c2_W1, c2_b1)
    S2 = _segsum(a2, eir, zr, zd, ones)
    t2 = _midA(h1, w32p[1])
    h2, a3 = _mid(S2, t2, deg1, l2_W, bps[1], c3_W1, c3_b1)
    S3 = _segsum(a3, eir, zr, zd, ones)
    t3 = _midA(h2, w32p[2])
    h3 = _post(S3, t3, deg1, l3_W, bps[2])
    return h3


# speedup vs baseline: 1.0926x; 1.0529x over previous
import jax
import jax.numpy as jnp
from jax import lax
from jax.experimental import pallas as pl
from jax.experimental.pallas import tpu as pltpu
from jax.experimental.pallas import tpu_sc as plsc

N = 10000
E = 320000
D = 128

NC = 2
NS = 16
HD = D // NC
CH = 128
NCHUNK = E // CH
RINGCH = 156
XBASE = NS * RINGCH
DEGW = 8

ROWS_BIG = 640
ROWS_LAST = N - 15 * ROWS_BIG

_sc_mesh = plsc.VectorSubcoreMesh(core_axis_name="c", subcore_axis_name="s")


def _segsum_body(with_deg, a_hbm, ei_hbm, zr_hbm, zd_hbm, ones_hbm,
                 rows_out, deg_out, src_v, dst_v, rb0, rb1, rb2, rb3,
                 sg0, sg1, sg2, sg3, ss0, ss1, ss2, ss3,
                 ones_v, acc_sh, deg_sh):
    c = lax.axis_index("c")
    s = lax.axis_index("s")

    @pl.when(s < NS - 1)
    def _():
        pltpu.sync_copy(zr_hbm, acc_sh.at[pl.ds(s * ROWS_BIG, ROWS_BIG)])
        if with_deg:
            pltpu.sync_copy(zd_hbm, deg_sh.at[pl.ds(s * ROWS_BIG, ROWS_BIG)])

    @pl.when(s == NS - 1)
    def _():
        pltpu.sync_copy(zr_hbm.at[pl.ds(0, ROWS_LAST)],
                        acc_sh.at[pl.ds(15 * ROWS_BIG, ROWS_LAST)])
        if with_deg:
            pltpu.sync_copy(zd_hbm.at[pl.ds(0, ROWS_LAST)],
                            deg_sh.at[pl.ds(15 * ROWS_BIG, ROWS_LAST)])

    pltpu.sync_copy(ei_hbm.at[0].at[pl.ds(s * RINGCH, RINGCH)],
                    src_v.at[pl.ds(0, RINGCH)])
    pltpu.sync_copy(ei_hbm.at[1].at[pl.ds(s * RINGCH, RINGCH)],
                    dst_v.at[pl.ds(0, RINGCH)])

    @pl.when(s < NCHUNK - XBASE)
    def _():
        pltpu.sync_copy(ei_hbm.at[0].at[pl.ds(XBASE + s, 1)],
                        src_v.at[pl.ds(RINGCH, 1)])
        pltpu.sync_copy(ei_hbm.at[1].at[pl.ds(XBASE + s, 1)],
                        dst_v.at[pl.ds(RINGCH, 1)])

    if with_deg:
        pltpu.sync_copy(ones_hbm, ones_v)

    plsc.subcore_barrier()

    table = a_hbm.at[c]
    RB = (rb0, rb1, rb2, rb3)
    SG = (sg0, sg1, sg2, sg3)
    SS = (ss0, ss1, ss2, ss3)

    def gather_start(g, j):
        pltpu.async_copy(table.at[src_v.at[g]], RB[j], SG[j])

    def gather_wait(g, j):
        pltpu.make_async_copy(table.at[src_v.at[g]], RB[j], SG[j]).wait()

    def scat_start(g, j):
        pltpu.async_copy(RB[j], acc_sh.at[dst_v.at[g]], SS[j], add=True)
        if with_deg:
            @pl.when(c == 0)
            def _():
                pltpu.sync_copy(ones_v, deg_sh.at[dst_v.at[g]], add=True)

    def scat_wait(g, j):
        pltpu.make_async_copy(RB[j], acc_sh.at[dst_v.at[g]], SS[j]).wait()

    def ring(i, carry):
        for j in range(4):
            g = i * 4 + j

            @pl.when(i > 0)
            def _():
                scat_wait(g - 4, j)
            gather_start(g, j)
            jm = (j + 2) % 4
            if j < 2:
                @pl.when(i > 0)
                def _():
                    gather_wait(g - 2, jm)
                    scat_start(g - 2, jm)
            else:
                gather_wait(g - 2, jm)
                scat_start(g - 2, jm)
        return carry

    lax.fori_loop(0, RINGCH // 4, ring, 0)

    for g in (RINGCH - 2, RINGCH - 1):
        gather_wait(g, g % 4)
        scat_start(g, g % 4)
    for g in range(RINGCH - 4, RINGCH):
        scat_wait(g, g % 4)

    @pl.when(s < NCHUNK - XBASE)
    def _():
        pltpu.async_copy(table.at[src_v.at[RINGCH]], rb0, sg0).wait()
        pltpu.sync_copy(rb0, acc_sh.at[dst_v.at[RINGCH]], add=True)
        if with_deg:
            @pl.when(c == 0)
            def _():
                pltpu.sync_copy(ones_v, deg_sh.at[dst_v.at[RINGCH]],
                                add=True)

    plsc.subcore_barrier()

    @pl.when(s < NS - 1)
    def _():
        r0 = s * ROWS_BIG
        pltpu.sync_copy(acc_sh.at[pl.ds(r0, ROWS_BIG)],
                        rows_out.at[pl.ds(r0, ROWS_BIG), pl.ds(c * HD, HD)])
        if with_deg:
            @pl.when(c == 0)
            def _():
                pltpu.sync_copy(deg_sh.at[pl.ds(r0, ROWS_BIG)],
                                deg_out.at[pl.ds(r0, ROWS_BIG)])

    @pl.when(s == NS - 1)
    def _():
        r0 = 15 * ROWS_BIG
        pltpu.sync_copy(acc_sh.at[pl.ds(r0, ROWS_LAST)],
                        rows_out.at[pl.ds(r0, ROWS_LAST), pl.ds(c * HD, HD)])
        if with_deg:
            @pl.when(c == 0)
            def _():
                pltpu.sync_copy(deg_sh.at[pl.ds(r0, ROWS_LAST)],
                                deg_out.at[pl.ds(r0, ROWS_LAST)])


def _make_segsum(with_deg):
    out_type = [jax.ShapeDtypeStruct((N, D), jnp.float32)]
    if with_deg:
        out_type.append(jax.ShapeDtypeStruct((N, DEGW), jnp.float32))
    scratch = [
        pltpu.VMEM((RINGCH + 1, CH), jnp.int32),
        pltpu.VMEM((RINGCH + 1, CH), jnp.int32),
        pltpu.VMEM((CH, HD), jnp.float32),
        pltpu.VMEM((CH, HD), jnp.float32),
        pltpu.VMEM((CH, HD), jnp.float32),
        pltpu.VMEM((CH, HD), jnp.float32),
        pltpu.SemaphoreType.DMA,
        pltpu.SemaphoreType.DMA,
        pltpu.SemaphoreType.DMA,
        pltpu.SemaphoreType.DMA,
        pltpu.SemaphoreType.DMA,
        pltpu.SemaphoreType.DMA,
        pltpu.SemaphoreType.DMA,
        pltpu.SemaphoreType.DMA,
        pltpu.VMEM((CH, DEGW), jnp.float32),
        pltpu.VMEM_SHARED((N, HD), jnp.float32),
        pltpu.VMEM_SHARED((N, DEGW), jnp.float32),
    ]

    def body(a_hbm, ei_hbm, zr_hbm, zd_hbm, ones_hbm, *refs):
        if with_deg:
            rows_out, deg_out = refs[0], refs[1]
            rest = refs[2:]
        else:
            rows_out, deg_out = refs[0], None
            rest = refs[1:]
        _segsum_body(with_deg, a_hbm, ei_hbm, zr_hbm, zd_hbm,
                     ones_hbm, rows_out, deg_out, *rest)

    return pl.kernel(body,
                     out_type=tuple(out_type) if with_deg else out_type[0],
                     mesh=_sc_mesh, scratch_types=scratch,
                     compiler_params=pltpu.CompilerParams(
                         use_tc_tiling_on_sc=False))


_segsum_deg = _make_segsum(True)
_segsum = _make_segsum(False)


_RB = 1000
_GRID = N // _RB


def _fold_body(w2_ref, w3_ref, b3_ref, lw_ref, lb_ref, w32p_ref, bp_ref):
    lw = lw_ref[0]
    w32p_ref[0, :, :D] = jnp.dot(w3_ref[0], lw,
                                 preferred_element_type=jnp.float32)
    w32p_ref[0, :, D:] = jnp.dot(w2_ref[0], lw,
                                 preferred_element_type=jnp.float32)
    bp_ref[0] = (jnp.dot(b3_ref[0], lw, preferred_element_type=jnp.float32)
                 + lb_ref[0])


def _fold(W2s, W3s, b3s, lWs, lbs):
    return pl.pallas_call(
        _fold_body,
        grid=(3,),
        in_specs=[pl.BlockSpec((1, D, D), lambda i: (i, 0, 0)),
                  pl.BlockSpec((1, D, D), lambda i: (i, 0, 0)),
                  pl.BlockSpec((1, 1, D), lambda i: (i, 0, 0)),
                  pl.BlockSpec((1, D, D), lambda i: (i, 0, 0)),
                  pl.BlockSpec((1, 1, D), lambda i: (i, 0, 0))],
        out_specs=[pl.BlockSpec((1, D, 2 * D), lambda i: (i, 0, 0)),
                   pl.BlockSpec((1, 1, D), lambda i: (i, 0, 0))],
        out_shape=[jax.ShapeDtypeStruct((3, D, 2 * D), jnp.float32),
                   jax.ShapeDtypeStruct((3, 1, D), jnp.float32)],
    )(W2s, W3s, b3s, lWs, lbs)


def _midA_body(x_ref, w_ref, o_ref):
    o_ref[...] = jnp.dot(x_ref[...], w_ref[...],
                         preferred_element_type=jnp.float32)


def _midA(x, w32p):
    return pl.pallas_call(
        _midA_body,
        grid=(_GRID,),
        in_specs=[pl.BlockSpec((_RB, D), lambda i: (i, 0)),
                  pl.BlockSpec((D, 2 * D), lambda i: (0, 0))],
        out_specs=pl.BlockSpec((_RB, 2 * D), lambda i: (i, 0)),
        out_shape=jax.ShapeDtypeStruct((N, 2 * D), jnp.float32),
    )(x, w32p)


def _split_cols(t, o_ref):
    o_ref[0] = t[:, :HD]
    o_ref[1] = t[:, HD:]


def _pre_body(x_ref, w_ref, b_ref, o_ref):
    t = (jnp.dot(x_ref[...], w_ref[...],
                 preferred_element_type=jnp.float32) + b_ref[...])
    _split_cols(t, o_ref)


def _pre(x, W1, b1):
    return pl.pallas_call(
        _pre_body,
        grid=(_GRID,),
        in_specs=[pl.BlockSpec((_RB, D), lambda i: (i, 0)),
                  pl.BlockSpec((D, D), lambda i: (0, 0)),
                  pl.BlockSpec((1, D), lambda i: (0, 0))],
        out_specs=pl.BlockSpec((NC, _RB, HD), lambda i: (0, i, 0)),
        out_shape=jax.ShapeDtypeStruct((NC, N, HD), jnp.float32),
    )(x, W1, b1.reshape(1, D))


def _mid_body(s_ref, t_ref, deg_ref, lw_ref, bp_ref, w1n_ref, b1n_ref,
              h_ref, a_ref):
    t32 = t_ref[...]
    d = deg_ref[...]
    t = jnp.dot(s_ref[...], lw_ref[...], preferred_element_type=jnp.float32)
    t = t + t32[:, :D] - d * t32[:, D:]
    h = jnp.maximum(t + bp_ref[...], 0.0)
    h_ref[...] = h
    a = (jnp.dot(h, w1n_ref[...], preferred_element_type=jnp.float32)
         + b1n_ref[...])
    _split_cols(a, a_ref)


def _mid(S, t32, deg1, lW, bp, W1n, b1n):
    return pl.pallas_call(
        _mid_body,
        grid=(_GRID,),
        in_specs=[pl.BlockSpec((_RB, D), lambda i: (i, 0)),
                  pl.BlockSpec((_RB, 2 * D), lambda i: (i, 0)),
                  pl.BlockSpec((_RB, 1), lambda i: (i, 0)),
                  pl.BlockSpec((D, D), lambda i: (0, 0)),
                  pl.BlockSpec((1, D), lambda i: (0, 0)),
                  pl.BlockSpec((D, D), lambda i: (0, 0)),
                  pl.BlockSpec((1, D), lambda i: (0, 0))],
        out_specs=[pl.BlockSpec((_RB, D), lambda i: (i, 0)),
                   pl.BlockSpec((NC, _RB, HD), lambda i: (0, i, 0))],
        out_shape=[jax.ShapeDtypeStruct((N, D), jnp.float32),
                   jax.ShapeDtypeStruct((NC, N, HD), jnp.float32)],
    )(S, t32, deg1, lW, bp, W1n, b1n.reshape(1, D))


def _post_body(s_ref, t_ref, deg_ref, lw_ref, bp_ref, h_ref):
    t32 = t_ref[...]
    d = deg_ref[...]
    t = jnp.dot(s_ref[...], lw_ref[...], preferred_element_type=jnp.float32)
    t = t + t32[:, :D] - d * t32[:, D:]
    h_ref[...] = jnp.maximum(t + bp_ref[...], 0.0)


def _post(S, t32, deg1, lW, bp):
    return pl.pallas_call(
        _post_body,
        grid=(_GRID,),
        in_specs=[pl.BlockSpec((_RB, D), lambda i: (i, 0)),
                  pl.BlockSpec((_RB, 2 * D), lambda i: (i, 0)),
                  pl.BlockSpec((_RB, 1), lambda i: (i, 0)),
                  pl.BlockSpec((D, D), lambda i: (0, 0)),
                  pl.BlockSpec((1, D), lambda i: (0, 0))],
        out_specs=pl.BlockSpec((_RB, D), lambda i: (i, 0)),
        out_shape=jax.ShapeDtypeStruct((N, D), jnp.float32),
    )(S, t32, deg1, lW, bp)


def kernel(x, edge_index, c1_W1, c1_b1, c1_W2, c1_W3, c1_b3, l1_W, l1_b,
           c2_W1, c2_b1, c2_W2, c2_W3, c2_b3, l2_W, l2_b,
           c3_W1, c3_b1, c3_W2, c3_W3, c3_b3, l3_W, l3_b):
    eir = edge_index.reshape(2, NCHUNK, CH)

    zr = jnp.zeros((ROWS_BIG, HD), jnp.float32)
    zd = jnp.zeros((ROWS_BIG, DEGW), jnp.float32)
    ones = jnp.ones((CH, DEGW), jnp.float32)

    w32p, bps = _fold(jnp.stack([c1_W2, c2_W2, c3_W2]),
                      jnp.stack([c1_W3, c2_W3, c3_W3]),
                      jnp.stack([c1_b3, c2_b3, c3_b3]).reshape(3, 1, D),
                      jnp.stack([l1_W, l2_W, l3_W]),
                      jnp.stack([l1_b, l2_b, l3_b]).reshape(3, 1, D))

    a1 = _pre(x, c1_W1, c1_b1)
    S1, deg = _segsum_deg(a1, eir, zr, zd, ones)
    t1 = _midA(x, w32p[0])
    deg1 = deg[:, :1]
    h1, a2 = _mid(S1, t1, deg1, l1_W, bps[0], c2_W1, c2_b1)
    S2 = _segsum(a2, eir, zr, zd, ones)
    t2 = _midA(h1, w32p[1])
    h2, a3 = _mid(S2, t2, deg1, l2_W, bps[1], c3_W1, c3_b1)
    S3 = _segsum(a3, eir, zr, zd, ones)
    t3 = _midA(h2, w32p[2])
    h3 = _post(S3, t3, deg1, l3_W, bps[2])
    return h3


# speedup vs baseline: 1.2541x; 1.1478x over previous
import jax
import jax.numpy as jnp
from jax import lax
from jax.experimental import pallas as pl
from jax.experimental.pallas import tpu as pltpu
from jax.experimental.pallas import tpu_sc as plsc

N = 10000
E = 320000
D = 128

NC = 2
NS = 16
NW = NC * NS
CH = 125
RINGCH = E // (NW * CH)
NCHUNK = NW * RINGCH
DEGW = 8

ROWS_BIG = 640
ROWS_LAST = N - 15 * ROWS_BIG

_sc_mesh = plsc.VectorSubcoreMesh(core_axis_name="c", subcore_axis_name="s")


def _segsum_body(with_deg, a_hbm, ei_hbm, zr_hbm, zd_hbm, ones_hbm,
                 rows_out, deg_out, src_v, dst_v, rb0, rb1, rb2, rb3,
                 sg0, sg1, sg2, sg3, ss0, ss1, ss2, ss3,
                 ones_v, acc_sh, deg_sh):
    c = lax.axis_index("c")
    s = lax.axis_index("s")

    @pl.when(s < NS - 1)
    def _():
        pltpu.sync_copy(zr_hbm, acc_sh.at[pl.ds(s * ROWS_BIG, ROWS_BIG)])
        if with_deg:
            pltpu.sync_copy(zd_hbm, deg_sh.at[pl.ds(s * ROWS_BIG, ROWS_BIG)])

    @pl.when(s == NS - 1)
    def _():
        pltpu.sync_copy(zr_hbm.at[pl.ds(0, ROWS_LAST)],
                        acc_sh.at[pl.ds(15 * ROWS_BIG, ROWS_LAST)])
        if with_deg:
            pltpu.sync_copy(zd_hbm.at[pl.ds(0, ROWS_LAST)],
                            deg_sh.at[pl.ds(15 * ROWS_BIG, ROWS_LAST)])

    w = c * NS + s
    pltpu.sync_copy(ei_hbm.at[0].at[pl.ds(w * RINGCH, RINGCH)], src_v)
    pltpu.sync_copy(ei_hbm.at[1].at[pl.ds(w * RINGCH, RINGCH)], dst_v)

    if with_deg:
        pltpu.sync_copy(ones_hbm, ones_v)

    plsc.subcore_barrier()

    table = a_hbm
    RB = (rb0, rb1, rb2, rb3)
    SG = (sg0, sg1, sg2, sg3)
    SS = (ss0, ss1, ss2, ss3)

    def gather_start(g, j):
        pltpu.async_copy(table.at[src_v.at[g]], RB[j], SG[j])

    def gather_wait(g, j):
        pltpu.make_async_copy(table.at[src_v.at[g]], RB[j], SG[j]).wait()

    def scat_start(g, j):
        pltpu.async_copy(RB[j], acc_sh.at[dst_v.at[g]], SS[j], add=True)
        if with_deg:
            pltpu.sync_copy(ones_v, deg_sh.at[dst_v.at[g]], add=True)

    def scat_wait(g, j):
        pltpu.make_async_copy(RB[j], acc_sh.at[dst_v.at[g]], SS[j]).wait()

    def ring(i, carry):
        for j in range(4):
            g = i * 4 + j

            @pl.when(i > 0)
            def _():
                scat_wait(g - 4, j)
            gather_start(g, j)
            jm = (j + 2) % 4
            if j < 2:
                @pl.when(i > 0)
                def _():
                    gather_wait(g - 2, jm)
                    scat_start(g - 2, jm)
            else:
                gather_wait(g - 2, jm)
                scat_start(g - 2, jm)
        return carry

    lax.fori_loop(0, RINGCH // 4, ring, 0)

    for g in (RINGCH - 2, RINGCH - 1):
        gather_wait(g, g % 4)
        scat_start(g, g % 4)
    for g in range(RINGCH - 4, RINGCH):
        scat_wait(g, g % 4)

    plsc.subcore_barrier()

    @pl.when(s < NS - 1)
    def _():
        r0 = s * ROWS_BIG
        pltpu.sync_copy(acc_sh.at[pl.ds(r0, ROWS_BIG)],
                        rows_out.at[c].at[pl.ds(r0, ROWS_BIG)])
        if with_deg:
            pltpu.sync_copy(deg_sh.at[pl.ds(r0, ROWS_BIG)],
                            deg_out.at[c].at[pl.ds(r0, ROWS_BIG)])

    @pl.when(s == NS - 1)
    def _():
        r0 = 15 * ROWS_BIG
        pltpu.sync_copy(acc_sh.at[pl.ds(r0, ROWS_LAST)],
                        rows_out.at[c].at[pl.ds(r0, ROWS_LAST)])
        if with_deg:
            pltpu.sync_copy(deg_sh.at[pl.ds(r0, ROWS_LAST)],
                            deg_out.at[c].at[pl.ds(r0, ROWS_LAST)])


def _make_segsum(with_deg):
    out_type = [jax.ShapeDtypeStruct((NC, N, D), jnp.bfloat16)]
    if with_deg:
        out_type.append(jax.ShapeDtypeStruct((NC, N, DEGW), jnp.float32))
    scratch = [
        pltpu.VMEM((RINGCH, CH), jnp.int32),
        pltpu.VMEM((RINGCH, CH), jnp.int32),
        pltpu.VMEM((CH, D), jnp.bfloat16),
        pltpu.VMEM((CH, D), jnp.bfloat16),
        pltpu.VMEM((CH, D), jnp.bfloat16),
        pltpu.VMEM((CH, D), jnp.bfloat16),
        pltpu.SemaphoreType.DMA,
        pltpu.SemaphoreType.DMA,
        pltpu.SemaphoreType.DMA,
        pltpu.SemaphoreType.DMA,
        pltpu.SemaphoreType.DMA,
        pltpu.SemaphoreType.DMA,
        pltpu.SemaphoreType.DMA,
        pltpu.SemaphoreType.DMA,
        pltpu.VMEM((CH, DEGW), jnp.float32),
        pltpu.VMEM_SHARED((N, D), jnp.bfloat16),
        pltpu.VMEM_SHARED((N, DEGW), jnp.float32),
    ]

    def body(a_hbm, ei_hbm, zr_hbm, zd_hbm, ones_hbm, *refs):
        if with_deg:
            rows_out, deg_out = refs[0], refs[1]
            rest = refs[2:]
        else:
            rows_out, deg_out = refs[0], None
            rest = refs[1:]
        _segsum_body(with_deg, a_hbm, ei_hbm, zr_hbm, zd_hbm,
                     ones_hbm, rows_out, deg_out, *rest)

    return pl.kernel(body,
                     out_type=tuple(out_type) if with_deg else out_type[0],
                     mesh=_sc_mesh, scratch_types=scratch,
                     compiler_params=pltpu.CompilerParams(
                         use_tc_tiling_on_sc=False))


_segsum_deg = _make_segsum(True)
_segsum = _make_segsum(False)


_RB = 1000
_GRID = N // _RB


def _fold_body(w2_ref, w3_ref, b3_ref, lw_ref, lb_ref, w32p_ref, bp_ref):
    lw = lw_ref[0]
    w32p_ref[0, :, :D] = jnp.dot(w3_ref[0], lw,
                                 preferred_element_type=jnp.float32)
    w32p_ref[0, :, D:] = jnp.dot(w2_ref[0], lw,
                                 preferred_element_type=jnp.float32)
    bp_ref[0] = (jnp.dot(b3_ref[0], lw, preferred_element_type=jnp.float32)
                 + lb_ref[0])


def _fold(W2s, W3s, b3s, lWs, lbs):
    return pl.pallas_call(
        _fold_body,
        grid=(3,),
        in_specs=[pl.BlockSpec((1, D, D), lambda i: (i, 0, 0)),
                  pl.BlockSpec((1, D, D), lambda i: (i, 0, 0)),
                  pl.BlockSpec((1, 1, D), lambda i: (i, 0, 0)),
                  pl.BlockSpec((1, D, D), lambda i: (i, 0, 0)),
                  pl.BlockSpec((1, 1, D), lambda i: (i, 0, 0))],
        out_specs=[pl.BlockSpec((1, D, 2 * D), lambda i: (i, 0, 0)),
                   pl.BlockSpec((1, 1, D), lambda i: (i, 0, 0))],
        out_shape=[jax.ShapeDtypeStruct((3, D, 2 * D), jnp.float32),
                   jax.ShapeDtypeStruct((3, 1, D), jnp.float32)],
    )(W2s, W3s, b3s, lWs, lbs)


def _midA_body(x_ref, w_ref, o_ref):
    o_ref[...] = jnp.dot(x_ref[...], w_ref[...],
                         preferred_element_type=jnp.float32)


def _midA(x, w32p):
    return pl.pallas_call(
        _midA_body,
        grid=(_GRID,),
        in_specs=[pl.BlockSpec((_RB, D), lambda i: (i, 0)),
                  pl.BlockSpec((D, 2 * D), lambda i: (0, 0))],
        out_specs=pl.BlockSpec((_RB, 2 * D), lambda i: (i, 0)),
        out_shape=jax.ShapeDtypeStruct((N, 2 * D), jnp.float32),
    )(x, w32p)


def _pre_body(x_ref, w_ref, b_ref, o_ref):
    t = (jnp.dot(x_ref[...], w_ref[...],
                 preferred_element_type=jnp.float32) + b_ref[...])
    o_ref[...] = t.astype(jnp.bfloat16)


def _pre(x, W1, b1):
    return pl.pallas_call(
        _pre_body,
        grid=(_GRID,),
        in_specs=[pl.BlockSpec((_RB, D), lambda i: (i, 0)),
                  pl.BlockSpec((D, D), lambda i: (0, 0)),
                  pl.BlockSpec((1, D), lambda i: (0, 0))],
        out_specs=pl.BlockSpec((_RB, D), lambda i: (i, 0)),
        out_shape=jax.ShapeDtypeStruct((N, D), jnp.bfloat16),
    )(x, W1, b1.reshape(1, D))


def _mid_body(s_ref, t_ref, deg_ref, lw_ref, bp_ref, w1n_ref, b1n_ref,
              h_ref, a_ref):
    sblk = (s_ref[0].astype(jnp.float32) + s_ref[1].astype(jnp.float32))
    t32 = t_ref[...]
    d = deg_ref[0] + deg_ref[1]
    t = jnp.dot(sblk, lw_ref[...], preferred_element_type=jnp.float32)
    t = t + t32[:, :D] - d * t32[:, D:]
    h = jnp.maximum(t + bp_ref[...], 0.0)
    h_ref[...] = h
    a = (jnp.dot(h, w1n_ref[...], preferred_element_type=jnp.float32)
         + b1n_ref[...])
    a_ref[...] = a.astype(jnp.bfloat16)


def _mid(S, t32, deg1, lW, bp, W1n, b1n):
    return pl.pallas_call(
        _mid_body,
        grid=(_GRID,),
        in_specs=[pl.BlockSpec((NC, _RB, D), lambda i: (0, i, 0)),
                  pl.BlockSpec((_RB, 2 * D), lambda i: (i, 0)),
                  pl.BlockSpec((NC, _RB, 1), lambda i: (0, i, 0)),
                  pl.BlockSpec((D, D), lambda i: (0, 0)),
                  pl.BlockSpec((1, D), lambda i: (0, 0)),
                  pl.BlockSpec((D, D), lambda i: (0, 0)),
                  pl.BlockSpec((1, D), lambda i: (0, 0))],
        out_specs=[pl.BlockSpec((_RB, D), lambda i: (i, 0)),
                   pl.BlockSpec((_RB, D), lambda i: (i, 0))],
        out_shape=[jax.ShapeDtypeStruct((N, D), jnp.float32),
                   jax.ShapeDtypeStruct((N, D), jnp.bfloat16)],
    )(S, t32, deg1, lW, bp, W1n, b1n.reshape(1, D))


def _post_body(s_ref, t_ref, deg_ref, lw_ref, bp_ref, h_ref):
    sblk = (s_ref[0].astype(jnp.float32) + s_ref[1].astype(jnp.float32))
    t32 = t_ref[...]
    d = deg_ref[0] + deg_ref[1]
    t = jnp.dot(sblk, lw_ref[...], preferred_element_type=jnp.float32)
    t = t + t32[:, :D] - d * t32[:, D:]
    h_ref[...] = jnp.maximum(t + bp_ref[...], 0.0)


def _post(S, t32, deg1, lW, bp):
    return pl.pallas_call(
        _post_body,
        grid=(_GRID,),
        in_specs=[pl.BlockSpec((NC, _RB, D), lambda i: (0, i, 0)),
                  pl.BlockSpec((_RB, 2 * D), lambda i: (i, 0)),
                  pl.BlockSpec((NC, _RB, 1), lambda i: (0, i, 0)),
                  pl.BlockSpec((D, D), lambda i: (0, 0)),
                  pl.BlockSpec((1, D), lambda i: (0, 0))],
        out_specs=pl.BlockSpec((_RB, D), lambda i: (i, 0)),
        out_shape=jax.ShapeDtypeStruct((N, D), jnp.float32),
    )(S, t32, deg1, lW, bp)


def kernel(x, edge_index, c1_W1, c1_b1, c1_W2, c1_W3, c1_b3, l1_W, l1_b,
           c2_W1, c2_b1, c2_W2, c2_W3, c2_b3, l2_W, l2_b,
           c3_W1, c3_b1, c3_W2, c3_W3, c3_b3, l3_W, l3_b):
    eir = edge_index.reshape(2, NCHUNK, CH)

    zr = jnp.zeros((ROWS_BIG, D), jnp.bfloat16)
    zd = jnp.zeros((ROWS_BIG, DEGW), jnp.float32)
    ones = jnp.ones((CH, DEGW), jnp.float32)

    w32p, bps = _fold(jnp.stack([c1_W2, c2_W2, c3_W2]),
                      jnp.stack([c1_W3, c2_W3, c3_W3]),
                      jnp.stack([c1_b3, c2_b3, c3_b3]).reshape(3, 1, D),
                      jnp.stack([l1_W, l2_W, l3_W]),
                      jnp.stack([l1_b, l2_b, l3_b]).reshape(3, 1, D))

    a1 = _pre(x, c1_W1, c1_b1)
    S1, deg = _segsum_deg(a1, eir, zr, zd, ones)
    t1 = _midA(x, w32p[0])
    deg1 = deg[:, :, :1]
    h1, a2 = _mid(S1, t1, deg1, l1_W, bps[0], c2_W1, c2_b1)
    S2 = _segsum(a2, eir, zr, zd, ones)
    t2 = _midA(h1, w32p[1])
    h2, a3 = _mid(S2, t2, deg1, l2_W, bps[1], c3_W1, c3_b1)
    S3 = _segsum(a3, eir, zr, zd, ones)
    t3 = _midA(h2, w32p[2])
    h3 = _post(S3, t3, deg1, l3_W, bps[2])
    return h3


# speedup vs baseline: 1.3518x; 1.0779x over previous
import jax
import jax.numpy as jnp
from jax import lax
from jax.experimental import pallas as pl
from jax.experimental.pallas import tpu as pltpu
from jax.experimental.pallas import tpu_sc as plsc

N = 10000
E = 320000
D = 128

NC = 2
NS = 16
NW = NC * NS
CH = 125
RINGCH = E // (NW * CH)
NCHUNK = NW * RINGCH
DEGW = 8

ROWS_BIG = 640
ROWS_LAST = N - 15 * ROWS_BIG

_sc_mesh = plsc.VectorSubcoreMesh(core_axis_name="c", subcore_axis_name="s")


def _segsum_body(with_deg, a_hbm, ei_hbm, zr_hbm, zd_hbm, ones_hbm,
                 rows_out, deg_out, src_v, dst_v, rb0, rb1, rb2, rb3,
                 sg0, sg1, sg2, sg3, ss0, ss1, ss2, ss3,
                 ones_v, acc_sh, deg_sh):
    c = lax.axis_index("c")
    s = lax.axis_index("s")

    @pl.when(s < NS - 1)
    def _():
        pltpu.sync_copy(zr_hbm, acc_sh.at[pl.ds(s * ROWS_BIG, ROWS_BIG)])
        if with_deg:
            pltpu.sync_copy(zd_hbm, deg_sh.at[pl.ds(s * ROWS_BIG, ROWS_BIG)])

    @pl.when(s == NS - 1)
    def _():
        pltpu.sync_copy(zr_hbm.at[pl.ds(0, ROWS_LAST)],
                        acc_sh.at[pl.ds(15 * ROWS_BIG, ROWS_LAST)])
        if with_deg:
            pltpu.sync_copy(zd_hbm.at[pl.ds(0, ROWS_LAST)],
                            deg_sh.at[pl.ds(15 * ROWS_BIG, ROWS_LAST)])

    w = c * NS + s
    pltpu.sync_copy(ei_hbm.at[0].at[pl.ds(w * RINGCH, RINGCH)], src_v)
    pltpu.sync_copy(ei_hbm.at[1].at[pl.ds(w * RINGCH, RINGCH)], dst_v)

    if with_deg:
        pltpu.sync_copy(ones_hbm, ones_v)

    plsc.subcore_barrier()

    table = a_hbm
    RB = (rb0, rb1, rb2, rb3)
    SG = (sg0, sg1, sg2, sg3)
    SS = (ss0, ss1, ss2, ss3)

    def gather_start(g, j):
        pltpu.async_copy(table.at[src_v.at[g]], RB[j], SG[j])

    def gather_wait(g, j):
        pltpu.make_async_copy(table.at[src_v.at[g]], RB[j], SG[j]).wait()

    def scat_start(g, j):
        pltpu.async_copy(RB[j], acc_sh.at[dst_v.at[g]], SS[j], add=True)
        if with_deg:
            pltpu.sync_copy(ones_v, deg_sh.at[dst_v.at[g]], add=True)

    def scat_wait(g, j):
        pltpu.make_async_copy(RB[j], acc_sh.at[dst_v.at[g]], SS[j]).wait()

    def ring(i, carry):
        for j in range(4):
            g = i * 4 + j

            @pl.when(i > 0)
            def _():
                scat_wait(g - 4, j)
            gather_start(g, j)
            jm = (j + 2) % 4
            if j < 2:
                @pl.when(i > 0)
                def _():
                    gather_wait(g - 2, jm)
                    scat_start(g - 2, jm)
            else:
                gather_wait(g - 2, jm)
                scat_start(g - 2, jm)
        return carry

    lax.fori_loop(0, RINGCH // 4, ring, 0)

    for g in (RINGCH - 2, RINGCH - 1):
        gather_wait(g, g % 4)
        scat_start(g, g % 4)
    for g in range(RINGCH - 4, RINGCH):
        scat_wait(g, g % 4)

    plsc.subcore_barrier()

    @pl.when(s < NS - 1)
    def _():
        r0 = s * ROWS_BIG
        pltpu.sync_copy(acc_sh.at[pl.ds(r0, ROWS_BIG)],
                        rows_out.at[pl.ds(r0, ROWS_BIG), pl.ds(c * D, D)])
        if with_deg:
            pltpu.sync_copy(deg_sh.at[pl.ds(r0, ROWS_BIG)],
                            deg_out.at[c].at[pl.ds(r0, ROWS_BIG)])

    @pl.when(s == NS - 1)
    def _():
        r0 = 15 * ROWS_BIG
        pltpu.sync_copy(acc_sh.at[pl.ds(r0, ROWS_LAST)],
                        rows_out.at[pl.ds(r0, ROWS_LAST), pl.ds(c * D, D)])
        if with_deg:
            pltpu.sync_copy(deg_sh.at[pl.ds(r0, ROWS_LAST)],
                            deg_out.at[c].at[pl.ds(r0, ROWS_LAST)])


def _make_segsum(with_deg):
    out_type = [jax.ShapeDtypeStruct((N, NC * D), jnp.bfloat16)]
    if with_deg:
        out_type.append(jax.ShapeDtypeStruct((NC, N, DEGW), jnp.float32))
    scratch = [
        pltpu.VMEM((RINGCH, CH), jnp.int32),
        pltpu.VMEM((RINGCH, CH), jnp.int32),
        pltpu.VMEM((CH, D), jnp.bfloat16),
        pltpu.VMEM((CH, D), jnp.bfloat16),
        pltpu.VMEM((CH, D), jnp.bfloat16),
        pltpu.VMEM((CH, D), jnp.bfloat16),
        pltpu.SemaphoreType.DMA,
        pltpu.SemaphoreType.DMA,
        pltpu.SemaphoreType.DMA,
        pltpu.SemaphoreType.DMA,
        pltpu.SemaphoreType.DMA,
        pltpu.SemaphoreType.DMA,
        pltpu.SemaphoreType.DMA,
        pltpu.SemaphoreType.DMA,
        pltpu.VMEM((CH, DEGW), jnp.float32),
        pltpu.VMEM_SHARED((N, D), jnp.bfloat16),
        pltpu.VMEM_SHARED((N, DEGW), jnp.float32),
    ]

    def body(a_hbm, ei_hbm, zr_hbm, zd_hbm, ones_hbm, *refs):
        if with_deg:
            rows_out, deg_out = refs[0], refs[1]
            rest = refs[2:]
        else:
            rows_out, deg_out = refs[0], None
            rest = refs[1:]
        _segsum_body(with_deg, a_hbm, ei_hbm, zr_hbm, zd_hbm,
                     ones_hbm, rows_out, deg_out, *rest)

    return pl.kernel(body,
                     out_type=tuple(out_type) if with_deg else out_type[0],
                     mesh=_sc_mesh, scratch_types=scratch,
                     compiler_params=pltpu.CompilerParams(
                         use_tc_tiling_on_sc=False))


_segsum_deg = _make_segsum(True)
_segsum = _make_segsum(False)


_RB = 1000
_GRID = N // _RB


def _fold_body(w2_ref, w3_ref, b3_ref, lw_ref, lb_ref, w32p_ref, bp_ref):
    lw = lw_ref[0]
    w32p_ref[0, :, :D] = jnp.dot(w3_ref[0], lw,
                                 preferred_element_type=jnp.float32)
    w32p_ref[0, :, D:] = jnp.dot(w2_ref[0], lw,
                                 preferred_element_type=jnp.float32)
    bp_ref[0] = (jnp.dot(b3_ref[0], lw, preferred_element_type=jnp.float32)
                 + lb_ref[0])


def _fold(W2s, W3s, b3s, lWs, lbs):
    return pl.pallas_call(
        _fold_body,
        grid=(3,),
        in_specs=[pl.BlockSpec((1, D, D), lambda i: (i, 0, 0)),
                  pl.BlockSpec((1, D, D), lambda i: (i, 0, 0)),
                  pl.BlockSpec((1, 1, D), lambda i: (i, 0, 0)),
                  pl.BlockSpec((1, D, D), lambda i: (i, 0, 0)),
                  pl.BlockSpec((1, 1, D), lambda i: (i, 0, 0))],
        out_specs=[pl.BlockSpec((1, D, 2 * D), lambda i: (i, 0, 0)),
                   pl.BlockSpec((1, 1, D), lambda i: (i, 0, 0))],
        out_shape=[jax.ShapeDtypeStruct((3, D, 2 * D), jnp.float32),
                   jax.ShapeDtypeStruct((3, 1, D), jnp.float32)],
    )(W2s, W3s, b3s, lWs, lbs)


def _midA_body(x_ref, w_ref, o_ref):
    o_ref[...] = jnp.dot(x_ref[...], w_ref[...],
                         preferred_element_type=jnp.float32)


def _midA(x, w32p):
    return pl.pallas_call(
        _midA_body,
        grid=(_GRID,),
        in_specs=[pl.BlockSpec((_RB, D), lambda i: (i, 0)),
                  pl.BlockSpec((D, 2 * D), lambda i: (0, 0))],
        out_specs=pl.BlockSpec((_RB, 2 * D), lambda i: (i, 0)),
        out_shape=jax.ShapeDtypeStruct((N, 2 * D), jnp.float32),
    )(x, w32p)


def _pre_body(x_ref, w_ref, b_ref, o_ref):
    t = (jnp.dot(x_ref[...], w_ref[...],
                 preferred_element_type=jnp.float32) + b_ref[...])
    o_ref[...] = t.astype(jnp.bfloat16)


def _pre(x, W1, b1):
    return pl.pallas_call(
        _pre_body,
        grid=(_GRID,),
        in_specs=[pl.BlockSpec((_RB, D), lambda i: (i, 0)),
                  pl.BlockSpec((D, D), lambda i: (0, 0)),
                  pl.BlockSpec((1, D), lambda i: (0, 0))],
        out_specs=pl.BlockSpec((_RB, D), lambda i: (i, 0)),
        out_shape=jax.ShapeDtypeStruct((N, D), jnp.bfloat16),
    )(x, W1, b1.reshape(1, D))


def _mid_body(s_ref, t_ref, deg_ref, lw_ref, bp_ref, w1n_ref, b1n_ref,
              h_ref, a_ref):
    sblk = (s_ref[:, :D].astype(jnp.float32)
            + s_ref[:, D:].astype(jnp.float32))
    t32 = t_ref[...]
    d = deg_ref[0, :, :1] + deg_ref[1, :, :1]
    t = jnp.dot(sblk, lw_ref[...], preferred_element_type=jnp.float32)
    t = t + t32[:, :D] - d * t32[:, D:]
    h = jnp.maximum(t + bp_ref[...], 0.0)
    h_ref[...] = h
    a = (jnp.dot(h, w1n_ref[...], preferred_element_type=jnp.float32)
         + b1n_ref[...])
    a_ref[...] = a.astype(jnp.bfloat16)


def _mid(S, t32, deg1, lW, bp, W1n, b1n):
    return pl.pallas_call(
        _mid_body,
        grid=(_GRID,),
        in_specs=[pl.BlockSpec((_RB, NC * D), lambda i: (i, 0)),
                  pl.BlockSpec((_RB, 2 * D), lambda i: (i, 0)),
                  pl.BlockSpec((NC, _RB, DEGW), lambda i: (0, i, 0)),
                  pl.BlockSpec((D, D), lambda i: (0, 0)),
                  pl.BlockSpec((1, D), lambda i: (0, 0)),
                  pl.BlockSpec((D, D), lambda i: (0, 0)),
                  pl.BlockSpec((1, D), lambda i: (0, 0))],
        out_specs=[pl.BlockSpec((_RB, D), lambda i: (i, 0)),
                   pl.BlockSpec((_RB, D), lambda i: (i, 0))],
        out_shape=[jax.ShapeDtypeStruct((N, D), jnp.float32),
                   jax.ShapeDtypeStruct((N, D), jnp.bfloat16)],
    )(S, t32, deg1, lW, bp, W1n, b1n.reshape(1, D))


def _post_body(s_ref, t_ref, deg_ref, lw_ref, bp_ref, h_ref):
    sblk = (s_ref[:, :D].astype(jnp.float32)
            + s_ref[:, D:].astype(jnp.float32))
    t32 = t_ref[...]
    d = deg_ref[0, :, :1] + deg_ref[1, :, :1]
    t = jnp.dot(sblk, lw_ref[...], preferred_element_type=jnp.float32)
    t = t + t32[:, :D] - d * t32[:, D:]
    h_ref[...] = jnp.maximum(t + bp_ref[...], 0.0)


def _post(S, t32, deg1, lW, bp):
    return pl.pallas_call(
        _post_body,
        grid=(_GRID,),
        in_specs=[pl.BlockSpec((_RB, NC * D), lambda i: (i, 0)),
                  pl.BlockSpec((_RB, 2 * D), lambda i: (i, 0)),
                  pl.BlockSpec((NC, _RB, DEGW), lambda i: (0, i, 0)),
                  pl.BlockSpec((D, D), lambda i: (0, 0)),
                  pl.BlockSpec((1, D), lambda i: (0, 0))],
        out_specs=pl.BlockSpec((_RB, D), lambda i: (i, 0)),
        out_shape=jax.ShapeDtypeStruct((N, D), jnp.float32),
    )(S, t32, deg1, lW, bp)


def kernel(x, edge_index, c1_W1, c1_b1, c1_W2, c1_W3, c1_b3, l1_W, l1_b,
           c2_W1, c2_b1, c2_W2, c2_W3, c2_b3, l2_W, l2_b,
           c3_W1, c3_b1, c3_W2, c3_W3, c3_b3, l3_W, l3_b):
    eir = edge_index.reshape(2, NCHUNK, CH)

    zr = jnp.zeros((ROWS_BIG, D), jnp.bfloat16)
    zd = jnp.zeros((ROWS_BIG, DEGW), jnp.float32)
    ones = jnp.ones((CH, DEGW), jnp.float32)

    w32p, bps = _fold(jnp.stack([c1_W2, c2_W2, c3_W2]),
                      jnp.stack([c1_W3, c2_W3, c3_W3]),
                      jnp.stack([c1_b3, c2_b3, c3_b3]).reshape(3, 1, D),
                      jnp.stack([l1_W, l2_W, l3_W]),
                      jnp.stack([l1_b, l2_b, l3_b]).reshape(3, 1, D))

    a1 = _pre(x, c1_W1, c1_b1)
    S1, deg1 = _segsum_deg(a1, eir, zr, zd, ones)
    t1 = _midA(x, w32p[0])
    h1, a2 = _mid(S1, t1, deg1, l1_W, bps[0], c2_W1, c2_b1)
    S2 = _segsum(a2, eir, zr, zd, ones)
    t2 = _midA(h1, w32p[1])
    h2, a3 = _mid(S2, t2, deg1, l2_W, bps[1], c3_W1, c3_b1)
    S3 = _segsum(a3, eir, zr, zd, ones)
    t3 = _midA(h2, w32p[2])
    h3 = _post(S3, t3, deg1, l3_W, bps[2])
    return h3
